# baseline (speedup 1.0000x reference)
import functools

import jax
import jax.numpy as jnp
from jax import lax
from jax.experimental import pallas as pl
from jax.experimental.pallas import tpu as pltpu
from jax.experimental.pallas import tpu_sc as plsc

D = 300
DP = 304
S = 50
SP = 64
NC = 2
NS = 16
NW = NC * NS
L = 16

NCHUNK = DP // L
UNROLL = 5
G = S + 1


def _sc_body(pidx_hbm, hidx_hbm, table_hbm, out_hbm,
             ip0, ip1, ih0, ih1, pbuf0, pbuf1, hbuf0, hbuf1, orow0, orow1,
             si0, si1, sg0, sg1, so0, so1):
    rows_per_w = pidx_hbm.shape[0] // (NW * SP)
    wid = lax.axis_index("s") * NC + lax.axis_index("c")
    base = wid * rows_per_w
    inv = jnp.float32(1.0 / S)

    bufs = ((ip0, ih0, pbuf0, hbuf0, si0, sg0, orow0, so0),
            (ip1, ih1, pbuf1, hbuf1, si1, sg1, orow1, so1))

    def fire_idx(r, k):
        ip, ih, _, _, si, _, _, _ = bufs[k]
        o = pl.multiple_of((base + r) * SP, SP)
        pltpu.async_copy(pidx_hbm.at[pl.ds(o, SP)], ip, si)
        pltpu.async_copy(hidx_hbm.at[pl.ds(o, SP)], ih, si)

    def wait_idx(r, k):
        ip, ih, _, _, si, _, _, _ = bufs[k]
        o = pl.multiple_of((base + r) * SP, SP)
        pltpu.make_async_copy(pidx_hbm.at[pl.ds(o, SP)], ip, si).wait()
        pltpu.make_async_copy(hidx_hbm.at[pl.ds(o, SP)], ih, si).wait()

    def fire_gather(k):
        ip, ih, pb, hb, _, sg, _, _ = bufs[k]
        pltpu.async_copy(table_hbm.at[ip.at[pl.ds(0, G)]], pb, sg)
        pltpu.async_copy(table_hbm.at[ih.at[pl.ds(0, G)]], hb, sg)

    def wait_gather(k):
        ip, ih, pb, hb, _, sg, _, _ = bufs[k]
        pltpu.make_async_copy(table_hbm.at[ip.at[pl.ds(0, G)]], pb, sg).wait()
        pltpu.make_async_copy(table_hbm.at[ih.at[pl.ds(0, G)]], hb, sg).wait()

    def process(r, k):
        _, _, pb, hb, _, _, orow, so = bufs[k]

        @pl.when(r >= 2)
        def _():
            pltpu.make_async_copy(
                orow, out_hbm.at[pl.ds((base + r - 2) * 4, 4)], so).wait()

        for c in range(NCHUNK):
            off = c * L

            def acc(jj, accs):
                ap, ah = accs
                for u in range(UNROLL):
                    j = jj * UNROLL + u
                    ap = ap + pb[j, pl.ds(off, L)]
                    ah = ah + hb[j, pl.ds(off, L)]
                return ap, ah

            z = jnp.zeros((L,), jnp.float32)
            ps, hs = lax.fori_loop(0, S // UNROLL, acc, (z, z))
            pm = ps * inv
            hm = hs * inv
            orow[0, pl.ds(off, L)] = pm
            orow[1, pl.ds(off, L)] = hm
            orow[2, pl.ds(off, L)] = jnp.abs(pm - hm)
            orow[3, pl.ds(off, L)] = pm * hm
        pltpu.async_copy(orow, out_hbm.at[pl.ds((base + r) * 4, 4)], so)

    half = rows_per_w // 2

    def body(t, carry):
        r0 = 2 * t
        wait_idx(r0 + 1, 1)
        fire_gather(1)
        wait_gather(0)

        @pl.when(t < half - 1)
        def _():
            fire_idx(r0 + 2, 0)

        process(r0, 0)

        @pl.when(t < half - 1)
        def _():
            wait_idx(r0 + 2, 0)
            fire_gather(0)

        wait_gather(1)

        @pl.when(t < half - 1)
        def _():
            fire_idx(r0 + 3, 1)

        process(r0 + 1, 1)
        return carry

    fire_idx(0, 0)
    wait_idx(0, 0)
    fire_gather(0)
    fire_idx(1, 1)
    lax.fori_loop(0, half, body, 0)
    pltpu.make_async_copy(
        orow0, out_hbm.at[pl.ds((base + rows_per_w - 2) * 4, 4)], so0).wait()
    pltpu.make_async_copy(
        orow1, out_hbm.at[pl.ds((base + rows_per_w - 1) * 4, 4)], so1).wait()


@jax.jit
def kernel(premises, hypothesis, glove_table):
    b = premises.shape[0]
    pidx = jnp.pad(premises.astype(jnp.int32), ((0, 0), (0, SP - S)),
                   mode="edge").reshape(-1)
    hidx = jnp.pad(hypothesis.astype(jnp.int32), ((0, 0), (0, SP - S)),
                   mode="edge").reshape(-1)
    table = jnp.pad(glove_table, ((0, 0), (0, DP - D)))
    mesh = plsc.VectorSubcoreMesh(core_axis_name="c", subcore_axis_name="s")
    run = pl.kernel(
        _sc_body,
        mesh=mesh,
        out_type=jax.ShapeDtypeStruct((b * 4, DP), jnp.float32),
        scratch_types=[
            pltpu.VMEM((SP,), jnp.int32),
            pltpu.VMEM((SP,), jnp.int32),
            pltpu.VMEM((SP,), jnp.int32),
            pltpu.VMEM((SP,), jnp.int32),
            pltpu.VMEM((G, DP), jnp.float32),
            pltpu.VMEM((G, DP), jnp.float32),
            pltpu.VMEM((G, DP), jnp.float32),
            pltpu.VMEM((G, DP), jnp.float32),
            pltpu.VMEM((4, DP), jnp.float32),
            pltpu.VMEM((4, DP), jnp.float32),
            pltpu.SemaphoreType.DMA,
            pltpu.SemaphoreType.DMA,
            pltpu.SemaphoreType.DMA,
            pltpu.SemaphoreType.DMA,
            pltpu.SemaphoreType.DMA,
            pltpu.SemaphoreType.DMA,
        ],
        compiler_params=pltpu.CompilerParams(use_tc_tiling_on_sc=False),
    )
    out = run(pidx, hidx, table)
    return out.reshape(b, 4, DP)[:, :, :D].reshape(b, 4 * D)
